# Initial kernel scaffold; baseline (speedup 1.0000x reference)
import jax
import jax.numpy as jnp
from jax.experimental import pallas as pl


def kernel(rules, rules_weight, bias, global_to_local):
    raise NotImplementedError("write your pallas kernel here")


# speedup vs baseline: 579.4680x; 579.4680x over previous
import functools

import jax
import jax.numpy as jnp
from jax import lax
from jax.experimental import pallas as pl
from jax.experimental.pallas import tpu as pltpu
from jax.experimental.pallas import tpu_sc as plsc

NC = 2
NS = 16
NW = NC * NS
LANES = 16


def _wid():
    return lax.axis_index("s") * NC + lax.axis_index("c")


def _mesh():
    return plsc.VectorSubcoreMesh(core_axis_name="c", subcore_axis_name="s")


_PARAMS = pltpu.CompilerParams(needs_layout_passes=False)


def _round_up(x, m):
    return (x + m - 1) // m * m


def _make_build_combined(gpad, tcnt, nw_rows):
    @functools.partial(
        pl.kernel,
        out_type=jax.ShapeDtypeStruct((gpad,), jnp.float32),
        mesh=_mesh(),
        compiler_params=_PARAMS,
        scratch_types=[
            pltpu.VMEM((nw_rows,), jnp.float32),
            pltpu.VMEM((tcnt,), jnp.int32),
            pltpu.VMEM((tcnt,), jnp.float32),
        ],
    )
    def build(g2l_hbm, w_hbm, comb_hbm, w_v, g2l_v, comb_v):
        base = _wid() * tcnt
        pltpu.sync_copy(w_hbm, w_v)
        pltpu.sync_copy(g2l_hbm.at[pl.ds(base, tcnt)], g2l_v)

        def body(c, carry):
            idx = g2l_v[pl.ds(c * LANES, LANES)]
            comb_v[pl.ds(c * LANES, LANES)] = plsc.load_gather(w_v, [idx])
            return carry

        lax.fori_loop(0, tcnt // LANES, body, 0, unroll=4)
        pltpu.sync_copy(comb_v, comb_hbm.at[pl.ds(base, tcnt)])

    return build


def _make_aggregate(batch, hist, gpad, rows_per_tile, unroll):
    groups = rows_per_tile // LANES
    steps = hist // unroll

    @functools.partial(
        pl.kernel,
        out_type=jax.ShapeDtypeStruct((batch,), jnp.float32),
        mesh=_mesh(),
        compiler_params=_PARAMS,
        scratch_types=[
            pltpu.VMEM((gpad,), jnp.float32),
            pltpu.VMEM((LANES * hist,), jnp.int32),
            pltpu.VMEM((rows_per_tile,), jnp.float32),
            pltpu.VMEM((LANES,), jnp.float32),
        ],
    )
    def aggregate(rules_hbm, comb_hbm, bias_hbm, out_hbm,
                  comb_v, rules_v, out_v, bias_v):
        wid = _wid()
        row0 = wid * rows_per_tile
        pltpu.sync_copy(comb_hbm, comb_v)
        pltpu.sync_copy(bias_hbm, bias_v)
        offs = lax.iota(jnp.int32, LANES) * hist

        def group(g, carry):
            pltpu.sync_copy(
                rules_hbm.at[pl.ds((row0 + g * LANES) * hist, LANES * hist)],
                rules_v)

            def step(j, acc):
                base_l = j * unroll
                for u in range(unroll):
                    idx = plsc.load_gather(rules_v, [offs + (base_l + u)])
                    acc = acc + plsc.load_gather(comb_v, [idx])
                return acc

            acc = lax.fori_loop(0, steps, step,
                                jnp.zeros((LANES,), jnp.float32))
            out_v[pl.ds(g * LANES, LANES)] = acc + bias_v[...]
            return carry

        lax.fori_loop(0, groups, group, 0)
        pltpu.sync_copy(out_v, out_hbm.at[pl.ds(row0, rows_per_tile)])

    return aggregate


def kernel(rules, rules_weight, bias, global_to_local):
    batch, hist = rules.shape
    n_globals = global_to_local.shape[0]
    n_weight_rows = rules_weight.shape[0]

    tcnt = _round_up(-(-n_globals // NW), LANES)
    gpad = tcnt * NW
    nw_rows = _round_up(n_weight_rows, 8)

    g2l_pad = jnp.pad(global_to_local, (0, gpad - n_globals))
    w_flat = jnp.pad(rules_weight[:, 0], (0, nw_rows - n_weight_rows))
    bias_vec = jnp.broadcast_to(bias.reshape(1), (LANES,))
    rules_flat = rules.reshape(batch * hist)

    combined = _make_build_combined(gpad, tcnt, nw_rows)(g2l_pad, w_flat)

    rows_per_tile = batch // NW
    out = _make_aggregate(batch, hist, gpad, rows_per_tile, 8)(
        rules_flat, combined, bias_vec)
    return out.reshape(batch, 1)
